# baseline (speedup 1.0000x reference)
import functools
import jax
import jax.numpy as jnp
from jax import lax
from jax.experimental import pallas as pl
from jax.experimental.pallas import tpu as pltpu
from jax.experimental.pallas import tpu_sc as plsc

_N = 10000
_D = 256
_H = 256
_C = 64
_E = 160000
_L = 2

_NC = 2
_NS = 16
_HH = _H // 2
_EPT = _E // _NS
_CH = 128
_EPP = 10240
_NCHUNK = _EPP // _CH
_NP = 10240
_RPT = _NP // _NS
_ZCH = 64
_NZ = _RPT // _ZCH
_IG = 16
_NG = _NCHUNK // _IG

_SRC_T = (0, 1, 1, 2)
_DST_T = (1, 0, 2, 1)

_BN = 1024
_NB = _NP // _BN


def _sc_segment_sums(h_stack, src_all, dst_all, zeros_hbm):
    mesh = plsc.VectorSubcoreMesh(core_axis_name="c", subcore_axis_name="s")

    def body(h_ref, src_ref, dst_ref, z_ref, out_ref,
             sa_v, sb_v, da_v, db_v, rows_a, rows_b,
             isem, gsem_a, gsem_b, ssem_a, ssem_b):
        c = lax.axis_index("c")
        s = lax.axis_index("s")
        row0 = s * _RPT

        for r in range(4):
            tab = h_ref.at[_SRC_T[r]]
            pltpu.sync_copy(z_ref, _ACC[0].at[pl.ds(row0, _RPT)])
            plsc.subcore_barrier()

            bufs = ((sa_v, da_v, rows_a), (sb_v, db_v, rows_b))
            pltpu.sync_copy(src_ref.at[r, s, pl.ds(0, _IG)], sa_v)
            pltpu.sync_copy(dst_ref.at[r, s, pl.ds(0, _IG)], da_v)
            for g in range(_NG):
                s_cur, d_cur, _ = bufs[g % 2]
                s_nxt, d_nxt, _ = bufs[(g + 1) % 2]
                if g + 1 < _NG:
                    pltpu.async_copy(
                        src_ref.at[r, s, pl.ds((g + 1) * _IG, _IG)], s_nxt,
                        isem)
                    pltpu.async_copy(
                        dst_ref.at[r, s, pl.ds((g + 1) * _IG, _IG)], d_nxt,
                        isem)

                pltpu.async_copy(tab.at[c].at[s_cur.at[0]], rows_a, gsem_a)
                pltpu.async_copy(tab.at[c].at[s_cur.at[1]], rows_b, gsem_b)
                pltpu.make_async_copy(tab.at[c].at[s_cur.at[0]], rows_a,
                                      gsem_a).wait()
                pltpu.async_copy(rows_a, _ACC[0].at[d_cur.at[0]], ssem_a,
                                 add=True)
                pltpu.make_async_copy(tab.at[c].at[s_cur.at[1]], rows_b,
                                      gsem_b).wait()
                pltpu.async_copy(rows_b, _ACC[0].at[d_cur.at[1]], ssem_b,
                                 add=True)

                @pl.loop(1, _IG // 2)
                def _pair(p):
                    c0 = 2 * p
                    c1 = 2 * p + 1
                    pltpu.make_async_copy(rows_a, _ACC[0].at[d_cur.at[c0]],
                                          ssem_a).wait()
                    pltpu.async_copy(tab.at[c].at[s_cur.at[c0]], rows_a,
                                     gsem_a)
                    pltpu.make_async_copy(rows_b, _ACC[0].at[d_cur.at[c1]],
                                          ssem_b).wait()
                    pltpu.async_copy(tab.at[c].at[s_cur.at[c1]], rows_b,
                                     gsem_b)
                    pltpu.make_async_copy(tab.at[c].at[s_cur.at[c0]], rows_a,
                                          gsem_a).wait()
                    pltpu.async_copy(rows_a, _ACC[0].at[d_cur.at[c0]], ssem_a,
                                     add=True)
                    pltpu.make_async_copy(tab.at[c].at[s_cur.at[c1]], rows_b,
                                          gsem_b).wait()
                    pltpu.async_copy(rows_b, _ACC[0].at[d_cur.at[c1]], ssem_b,
                                     add=True)

                pltpu.make_async_copy(rows_a, _ACC[0].at[d_cur.at[_IG - 2]],
                                      ssem_a).wait()
                pltpu.make_async_copy(rows_b, _ACC[0].at[d_cur.at[_IG - 1]],
                                      ssem_b).wait()
                if g + 1 < _NG:
                    pltpu.make_async_copy(
                        src_ref.at[r, s, pl.ds((g + 1) * _IG, _IG)], s_nxt,
                        isem).wait()
                    pltpu.make_async_copy(
                        dst_ref.at[r, s, pl.ds((g + 1) * _IG, _IG)], d_nxt,
                        isem).wait()

            plsc.subcore_barrier()
            pltpu.sync_copy(_ACC[0].at[pl.ds(row0, _RPT)],
                            out_ref.at[r, c, pl.ds(row0, _RPT)])

    _ACC = [None]

    def body_wrap(h_ref, src_ref, dst_ref, z_ref, out_ref,
                  sa_v, sb_v, da_v, db_v, rows_a, rows_b, acc_sh,
                  isem, gsem_a, gsem_b, ssem_a, ssem_b):
        _ACC[0] = acc_sh
        body(h_ref, src_ref, dst_ref, z_ref, out_ref,
             sa_v, sb_v, da_v, db_v, rows_a, rows_b,
             isem, gsem_a, gsem_b, ssem_a, ssem_b)

    k = pl.kernel(
        body_wrap,
        out_type=jax.ShapeDtypeStruct((4, _NC, _NP, _HH), jnp.bfloat16),
        mesh=mesh,
        scratch_types=[
            pltpu.VMEM((_IG, _CH), jnp.int32),
            pltpu.VMEM((_IG, _CH), jnp.int32),
            pltpu.VMEM((_IG, _CH), jnp.int32),
            pltpu.VMEM((_IG, _CH), jnp.int32),
            pltpu.VMEM((_CH, _HH), jnp.bfloat16),
            pltpu.VMEM((_CH, _HH), jnp.bfloat16),
            pltpu.VMEM_SHARED((_NP, _HH), jnp.bfloat16),
            pltpu.SemaphoreType.DMA,
            pltpu.SemaphoreType.DMA,
            pltpu.SemaphoreType.DMA,
            pltpu.SemaphoreType.DMA,
            pltpu.SemaphoreType.DMA,
        ],
        compiler_params=pltpu.CompilerParams(
            needs_layout_passes=False, use_tc_tiling_on_sc=False),
        name="sc_segsum",
    )
    return k(h_stack, src_all, dst_all, zeros_hbm)


def _sc_counts(dst_all):
    mesh = plsc.VectorSubcoreMesh(core_axis_name="c", subcore_axis_name="s")

    def body(dst_ref, cnt_ref, dst_v, hist_v, sem):
        c = lax.axis_index("c")
        s = lax.axis_index("s")
        zeros16 = jnp.zeros((16,), jnp.float32)
        ones16 = jnp.ones((16,), jnp.float32)
        for r in range(4):
            @pl.when(c == r // 2)
            def _rel():
                @pl.loop(0, _NP // 16)
                def _hz(i):
                    hist_v[pl.ds(i * 16, 16)] = zeros16

                for g in range(_NG):
                    pltpu.sync_copy(dst_ref.at[r, s, pl.ds(g * _IG, _IG)],
                                    dst_v)

                    @pl.loop(0, _IG)
                    def _chunk(j):
                        for gg in range(_CH // 16):
                            idx = dst_v[j, pl.ds(gg * 16, 16)]
                            plsc.addupdate_scatter(hist_v, [idx], ones16)

                pltpu.sync_copy(hist_v,
                                cnt_ref.at[pl.ds((s * 4 + r) * _NP, _NP)])

    return pl.kernel(
        body,
        out_type=jax.ShapeDtypeStruct((_NS * 4 * _NP,), jnp.float32),
        mesh=mesh,
        scratch_types=[
            pltpu.VMEM((_IG, _CH), jnp.int32),
            pltpu.VMEM((_NP,), jnp.float32),
            pltpu.SemaphoreType.DMA,
        ],
        compiler_params=pltpu.CompilerParams(
            needs_layout_passes=False, use_tc_tiling_on_sc=False),
        name="sc_counts",
    )(dst_all)


def _tc_project(xs, Wps, bps):
    def body(x_ref, w_ref, b_ref, o_ref):
        o_ref[0, 0] = (
            jnp.dot(x_ref[0], w_ref[0], preferred_element_type=jnp.float32)
            + b_ref[0, 0]
        ).astype(jnp.bfloat16)

    return pl.pallas_call(
        body,
        grid=(3, _NB, _NC),
        in_specs=[
            pl.BlockSpec((1, _BN, _D), lambda t, i, c: (t, i, 0)),
            pl.BlockSpec((1, _D, _HH), lambda t, i, c: (t, 0, c)),
            pl.BlockSpec((1, 1, 1, _HH), lambda t, i, c: (t, c, 0, 0)),
        ],
        out_specs=pl.BlockSpec((1, 1, _BN, _HH), lambda t, i, c: (t, c, i, 0)),
        out_shape=jax.ShapeDtypeStruct((3, _NC, _NP, _HH), jnp.bfloat16),
        name="tc_project",
    )(xs, Wps, bps)


def _tc_root(h_stack, Wr_l, bl_l):
    def body(h_ref, wr_ref, bl_ref, o_ref):
        for r in range(4):
            t = _DST_T[r]
            o_ref[r] = (
                jnp.dot(h_ref[t, 0].astype(jnp.float32), wr_ref[r, :_HH, :],
                        preferred_element_type=jnp.float32)
                + jnp.dot(h_ref[t, 1].astype(jnp.float32), wr_ref[r, _HH:, :],
                          preferred_element_type=jnp.float32)
                + bl_ref[r]
            )

    return pl.pallas_call(
        body,
        grid=(_NB,),
        in_specs=[
            pl.BlockSpec((3, _NC, _BN, _HH), lambda i: (0, 0, i, 0)),
            pl.BlockSpec((4, _H, _H), lambda i: (0, 0, 0)),
            pl.BlockSpec((4, _H), lambda i: (0, 0)),
        ],
        out_specs=pl.BlockSpec((4, _BN, _H), lambda i: (0, i, 0)),
        out_shape=jax.ShapeDtypeStruct((4, _NP, _H), jnp.float32),
        name="tc_root",
    )(h_stack, Wr_l, bl_l)


def _tc_layer_combine(sums, cnt, root, Wl_l):
    def body(s_ref, c_ref, rt_ref, wl_ref, o_ref):
        cnts = jnp.sum(c_ref[...], axis=0)
        o = []
        for r in range(4):
            acc = jnp.dot(s_ref[r, 0].astype(jnp.float32), wl_ref[r, :_HH, :],
                          preferred_element_type=jnp.float32)
            acc = acc + jnp.dot(s_ref[r, 1].astype(jnp.float32),
                                wl_ref[r, _HH:, :],
                                preferred_element_type=jnp.float32)
            inv = 1.0 / jnp.maximum(cnts[r][:, None], 1.0)
            o.append(acc * inv + rt_ref[r])
        new = (jnp.maximum(o[1], 0.0),
               jnp.maximum((o[0] + o[3]) * 0.5, 0.0),
               jnp.maximum(o[2], 0.0))
        for t in range(3):
            o_ref[t, 0] = new[t][:, :_HH].astype(jnp.bfloat16)
            o_ref[t, 1] = new[t][:, _HH:].astype(jnp.bfloat16)

    return pl.pallas_call(
        body,
        grid=(_NB,),
        in_specs=[
            pl.BlockSpec((4, _NC, _BN, _HH), lambda i: (0, 0, i, 0)),
            pl.BlockSpec((_NS, 4, _BN), lambda i: (0, 0, i)),
            pl.BlockSpec((4, _BN, _H), lambda i: (0, i, 0)),
            pl.BlockSpec((4, _H, _H), lambda i: (0, 0, 0)),
        ],
        out_specs=pl.BlockSpec((3, _NC, _BN, _HH), lambda i: (0, 0, i, 0)),
        out_shape=jax.ShapeDtypeStruct((3, _NC, _NP, _HH), jnp.bfloat16),
        name="tc_layer_combine",
    )(sums, cnt, root, Wl_l)


def _tc_classifier(h_stack, Wc, bc):
    def body(h_ref, w_ref, b_ref, o_ref):
        o_ref[...] = (
            jnp.dot(h_ref[0, 0].astype(jnp.float32), w_ref[:_HH, :],
                    preferred_element_type=jnp.float32)
            + jnp.dot(h_ref[0, 1].astype(jnp.float32), w_ref[_HH:, :],
                      preferred_element_type=jnp.float32)
            + b_ref[...]
        )

    return pl.pallas_call(
        body,
        grid=(_NB,),
        in_specs=[
            pl.BlockSpec((1, _NC, _BN, _HH), lambda i: (0, 0, i, 0)),
            pl.BlockSpec((_H, _C), lambda i: (0, 0)),
            pl.BlockSpec((1, _C), lambda i: (0, 0)),
        ],
        out_specs=pl.BlockSpec((_BN, _C), lambda i: (i, 0)),
        out_shape=jax.ShapeDtypeStruct((_NP, _C), jnp.float32),
        name="tc_classifier",
    )(h_stack, Wc, bc.reshape(1, _C))


def kernel(x_lncRNA, x_miRNA, x_mRNA, edge_index_interacts,
           edge_index_rev_interacts, edge_index_regulates,
           edge_index_rev_regulates, Wp_lncRNA, bp_lncRNA, Wp_miRNA, bp_miRNA,
           Wp_mRNA, bp_mRNA, Wl, bl, Wr, Wc, bc):
    xs = jnp.stack([x_lncRNA, x_miRNA, x_mRNA])
    xs = jnp.pad(xs, ((0, 0), (0, _NP - _N), (0, 0)))
    Wps = jnp.stack([Wp_lncRNA, Wp_miRNA, Wp_mRNA])
    bps = jnp.stack([bp_lncRNA, bp_miRNA, bp_mRNA]).reshape(3, _NC, 1, _HH)

    eis = (edge_index_interacts, edge_index_rev_interacts,
           edge_index_regulates, edge_index_rev_regulates)
    pad_row = jnp.int32(_NP - 1)

    def _prep(row):
        row = row.reshape(_NS, _EPT)
        row = jnp.pad(row, ((0, 0), (0, _EPP - _EPT)), constant_values=pad_row)
        return row.reshape(_NS, _NCHUNK, _CH)

    src_all = jnp.stack([_prep(e[0]) for e in eis])
    dst_all = jnp.stack([_prep(e[1]) for e in eis])

    h = _tc_project(xs, Wps, bps)

    cnt = _sc_counts(dst_all).reshape(_NS, 4, _NP)
    zeros_hbm = jnp.zeros((_RPT, _HH), jnp.bfloat16)
    for l in range(_L):
        sums = _sc_segment_sums(h, src_all, dst_all, zeros_hbm)
        root = _tc_root(h, Wr[l], bl[l])
        h = _tc_layer_combine(sums, cnt, root, Wl[l])

    return _tc_classifier(h, Wc, bc)[:_N]


# speedup vs baseline: 1.1665x; 1.1665x over previous
import functools
import jax
import jax.numpy as jnp
from jax import lax
from jax.experimental import pallas as pl
from jax.experimental.pallas import tpu as pltpu
from jax.experimental.pallas import tpu_sc as plsc

_N = 10000
_D = 256
_H = 256
_C = 64
_E = 160000
_L = 2

_NC = 2
_NS = 16
_HH = _H // 2
_EPT = _E // _NS
_CH = 128
_EPP = 10240
_NCHUNK = _EPP // _CH
_NP = 10240
_RPT = _NP // _NS
_ZCH = 64
_NZ = _RPT // _ZCH
_IG = 16
_NG = _NCHUNK // _IG

_SRC_T = (0, 1, 1, 2)
_DST_T = (1, 0, 2, 1)

_BN = 1024
_NB = _NP // _BN


def _sc_segment_sums(h_stack, src_all, dst_all, zeros_hbm):
    mesh = plsc.VectorSubcoreMesh(core_axis_name="c", subcore_axis_name="s")
    NB = 8
    D = 4

    def body(h_ref, src_ref, dst_ref, z_ref, out_ref, *rest):
        src_v, dst_v = rest[0], rest[1]
        rows = rest[2:2 + NB]
        acc_sh = rest[2 + NB]
        gsem = rest[3 + NB:3 + 2 * NB]
        ssem = rest[3 + 2 * NB:3 + 3 * NB]
        c = lax.axis_index("c")
        s = lax.axis_index("s")
        row0 = s * _RPT

        for r in range(4):
            tab = h_ref.at[_SRC_T[r]]
            pltpu.sync_copy(z_ref, acc_sh.at[pl.ds(row0, _RPT)])
            pltpu.sync_copy(src_ref.at[r, s], src_v)
            pltpu.sync_copy(dst_ref.at[r, s], dst_v)
            plsc.subcore_barrier()

            def issue_g(ch, k):
                pltpu.async_copy(tab.at[c].at[src_v.at[ch]], rows[k],
                                 gsem[k])

            def wait_g(ch, k):
                pltpu.make_async_copy(tab.at[c].at[src_v.at[ch]], rows[k],
                                      gsem[k]).wait()

            def issue_s(ch, k):
                pltpu.async_copy(rows[k], acc_sh.at[dst_v.at[ch]], ssem[k],
                                 add=True)

            def drain_s(ch, k):
                pltpu.make_async_copy(rows[k], acc_sh.at[dst_v.at[ch]],
                                      ssem[k]).wait()

            for k in range(D):
                issue_g(k, k)
            for k in range(NB):
                kd = (k + D) % NB
                if k + D >= NB:
                    drain_s(k - D, kd)
                issue_g(k + D, kd)
                wait_g(k, k)
                issue_s(k, k)

            @pl.loop(1, _NCHUNK // NB - 1)
            def _octet(q):
                base = q * NB
                for k in range(NB):
                    ch = base + k
                    kd = (k + D) % NB
                    drain_s(ch - D, kd)
                    issue_g(ch + D, kd)
                    wait_g(ch, k)
                    issue_s(ch, k)

            base = _NCHUNK - NB
            for k in range(NB):
                ch = base + k
                if ch + D < _NCHUNK:
                    drain_s(ch - D, (ch + D) % NB)
                    issue_g(ch + D, (ch + D) % NB)
                wait_g(ch, k)
                issue_s(ch, k)
            for ch in range(_NCHUNK - D, _NCHUNK):
                drain_s(ch, ch % NB)
            for ch in range(_NCHUNK - NB, _NCHUNK - D):
                drain_s(ch, ch % NB)

            plsc.subcore_barrier()
            pltpu.sync_copy(acc_sh.at[pl.ds(row0, _RPT)],
                            out_ref.at[r, c, pl.ds(row0, _RPT)])

    scratch = [
        pltpu.VMEM((_NCHUNK, _CH), jnp.int32),
        pltpu.VMEM((_NCHUNK, _CH), jnp.int32),
    ]
    scratch += [pltpu.VMEM((_CH, _HH), jnp.bfloat16) for _ in range(NB)]
    scratch += [pltpu.VMEM_SHARED((_NP, _HH), jnp.bfloat16)]
    scratch += [pltpu.SemaphoreType.DMA for _ in range(2 * NB)]

    k = pl.kernel(
        body,
        out_type=jax.ShapeDtypeStruct((4, _NC, _NP, _HH), jnp.bfloat16),
        mesh=mesh,
        scratch_types=scratch,
        compiler_params=pltpu.CompilerParams(
            needs_layout_passes=False, use_tc_tiling_on_sc=False),
        name="sc_segsum",
    )
    return k(h_stack, src_all, dst_all, zeros_hbm)


def _sc_counts(dst_all):
    mesh = plsc.VectorSubcoreMesh(core_axis_name="c", subcore_axis_name="s")

    def body(dst_ref, cnt_ref, dst_v, hist_v, sem):
        c = lax.axis_index("c")
        s = lax.axis_index("s")
        zeros16 = jnp.zeros((16,), jnp.float32)
        ones16 = jnp.ones((16,), jnp.float32)
        for r in range(4):
            @pl.when(c == r // 2)
            def _rel():
                @pl.loop(0, _NP // 16)
                def _hz(i):
                    hist_v[pl.ds(i * 16, 16)] = zeros16

                for g in range(_NG):
                    pltpu.sync_copy(dst_ref.at[r, s, pl.ds(g * _IG, _IG)],
                                    dst_v)

                    @pl.loop(0, _IG)
                    def _chunk(j):
                        for gg in range(_CH // 16):
                            idx = dst_v[j, pl.ds(gg * 16, 16)]
                            plsc.addupdate_scatter(hist_v, [idx], ones16)

                pltpu.sync_copy(hist_v,
                                cnt_ref.at[pl.ds((s * 4 + r) * _NP, _NP)])

    return pl.kernel(
        body,
        out_type=jax.ShapeDtypeStruct((_NS * 4 * _NP,), jnp.float32),
        mesh=mesh,
        scratch_types=[
            pltpu.VMEM((_IG, _CH), jnp.int32),
            pltpu.VMEM((_NP,), jnp.float32),
            pltpu.SemaphoreType.DMA,
        ],
        compiler_params=pltpu.CompilerParams(
            needs_layout_passes=False, use_tc_tiling_on_sc=False),
        name="sc_counts",
    )(dst_all)


def _tc_project(xs, Wps, bps):
    def body(x_ref, w_ref, b_ref, o_ref):
        o_ref[0, 0] = (
            jnp.dot(x_ref[0], w_ref[0], preferred_element_type=jnp.float32)
            + b_ref[0, 0]
        ).astype(jnp.bfloat16)

    return pl.pallas_call(
        body,
        grid=(3, _NB, _NC),
        in_specs=[
            pl.BlockSpec((1, _BN, _D), lambda t, i, c: (t, i, 0)),
            pl.BlockSpec((1, _D, _HH), lambda t, i, c: (t, 0, c)),
            pl.BlockSpec((1, 1, 1, _HH), lambda t, i, c: (t, c, 0, 0)),
        ],
        out_specs=pl.BlockSpec((1, 1, _BN, _HH), lambda t, i, c: (t, c, i, 0)),
        out_shape=jax.ShapeDtypeStruct((3, _NC, _NP, _HH), jnp.bfloat16),
        name="tc_project",
    )(xs, Wps, bps)


def _tc_root(h_stack, Wr_l, bl_l):
    def body(h_ref, wr_ref, bl_ref, o_ref):
        for r in range(4):
            t = _DST_T[r]
            o_ref[r] = (
                jnp.dot(h_ref[t, 0].astype(jnp.float32), wr_ref[r, :_HH, :],
                        preferred_element_type=jnp.float32)
                + jnp.dot(h_ref[t, 1].astype(jnp.float32), wr_ref[r, _HH:, :],
                          preferred_element_type=jnp.float32)
                + bl_ref[r]
            )

    return pl.pallas_call(
        body,
        grid=(_NB,),
        in_specs=[
            pl.BlockSpec((3, _NC, _BN, _HH), lambda i: (0, 0, i, 0)),
            pl.BlockSpec((4, _H, _H), lambda i: (0, 0, 0)),
            pl.BlockSpec((4, _H), lambda i: (0, 0)),
        ],
        out_specs=pl.BlockSpec((4, _BN, _H), lambda i: (0, i, 0)),
        out_shape=jax.ShapeDtypeStruct((4, _NP, _H), jnp.float32),
        name="tc_root",
    )(h_stack, Wr_l, bl_l)


def _tc_layer_combine(sums, cnt, root, Wl_l):
    def body(s_ref, c_ref, rt_ref, wl_ref, o_ref):
        cnts = jnp.sum(c_ref[...], axis=0)
        o = []
        for r in range(4):
            acc = jnp.dot(s_ref[r, 0].astype(jnp.float32), wl_ref[r, :_HH, :],
                          preferred_element_type=jnp.float32)
            acc = acc + jnp.dot(s_ref[r, 1].astype(jnp.float32),
                                wl_ref[r, _HH:, :],
                                preferred_element_type=jnp.float32)
            inv = 1.0 / jnp.maximum(cnts[r][:, None], 1.0)
            o.append(acc * inv + rt_ref[r])
        new = (jnp.maximum(o[1], 0.0),
               jnp.maximum((o[0] + o[3]) * 0.5, 0.0),
               jnp.maximum(o[2], 0.0))
        for t in range(3):
            o_ref[t, 0] = new[t][:, :_HH].astype(jnp.bfloat16)
            o_ref[t, 1] = new[t][:, _HH:].astype(jnp.bfloat16)

    return pl.pallas_call(
        body,
        grid=(_NB,),
        in_specs=[
            pl.BlockSpec((4, _NC, _BN, _HH), lambda i: (0, 0, i, 0)),
            pl.BlockSpec((_NS, 4, _BN), lambda i: (0, 0, i)),
            pl.BlockSpec((4, _BN, _H), lambda i: (0, i, 0)),
            pl.BlockSpec((4, _H, _H), lambda i: (0, 0, 0)),
        ],
        out_specs=pl.BlockSpec((3, _NC, _BN, _HH), lambda i: (0, 0, i, 0)),
        out_shape=jax.ShapeDtypeStruct((3, _NC, _NP, _HH), jnp.bfloat16),
        name="tc_layer_combine",
    )(sums, cnt, root, Wl_l)


def _tc_classifier(h_stack, Wc, bc):
    def body(h_ref, w_ref, b_ref, o_ref):
        o_ref[...] = (
            jnp.dot(h_ref[0, 0].astype(jnp.float32), w_ref[:_HH, :],
                    preferred_element_type=jnp.float32)
            + jnp.dot(h_ref[0, 1].astype(jnp.float32), w_ref[_HH:, :],
                      preferred_element_type=jnp.float32)
            + b_ref[...]
        )

    return pl.pallas_call(
        body,
        grid=(_NB,),
        in_specs=[
            pl.BlockSpec((1, _NC, _BN, _HH), lambda i: (0, 0, i, 0)),
            pl.BlockSpec((_H, _C), lambda i: (0, 0)),
            pl.BlockSpec((1, _C), lambda i: (0, 0)),
        ],
        out_specs=pl.BlockSpec((_BN, _C), lambda i: (i, 0)),
        out_shape=jax.ShapeDtypeStruct((_NP, _C), jnp.float32),
        name="tc_classifier",
    )(h_stack, Wc, bc.reshape(1, _C))


def kernel(x_lncRNA, x_miRNA, x_mRNA, edge_index_interacts,
           edge_index_rev_interacts, edge_index_regulates,
           edge_index_rev_regulates, Wp_lncRNA, bp_lncRNA, Wp_miRNA, bp_miRNA,
           Wp_mRNA, bp_mRNA, Wl, bl, Wr, Wc, bc):
    xs = jnp.stack([x_lncRNA, x_miRNA, x_mRNA])
    xs = jnp.pad(xs, ((0, 0), (0, _NP - _N), (0, 0)))
    Wps = jnp.stack([Wp_lncRNA, Wp_miRNA, Wp_mRNA])
    bps = jnp.stack([bp_lncRNA, bp_miRNA, bp_mRNA]).reshape(3, _NC, 1, _HH)

    eis = (edge_index_interacts, edge_index_rev_interacts,
           edge_index_regulates, edge_index_rev_regulates)
    pad_row = jnp.int32(_NP - 1)

    def _prep(row):
        row = row.reshape(_NS, _EPT)
        row = jnp.pad(row, ((0, 0), (0, _EPP - _EPT)), constant_values=pad_row)
        return row.reshape(_NS, _NCHUNK, _CH)

    src_all = jnp.stack([_prep(e[0]) for e in eis])
    dst_all = jnp.stack([_prep(e[1]) for e in eis])

    h = _tc_project(xs, Wps, bps)

    cnt = _sc_counts(dst_all).reshape(_NS, 4, _NP)
    zeros_hbm = jnp.zeros((_RPT, _HH), jnp.bfloat16)
    for l in range(_L):
        sums = _sc_segment_sums(h, src_all, dst_all, zeros_hbm)
        root = _tc_root(h, Wr[l], bl[l])
        h = _tc_layer_combine(sums, cnt, root, Wl[l])

    return _tc_classifier(h, Wc, bc)[:_N]


# speedup vs baseline: 1.1751x; 1.0073x over previous
import functools
import jax
import jax.numpy as jnp
from jax import lax
from jax.experimental import pallas as pl
from jax.experimental.pallas import tpu as pltpu
from jax.experimental.pallas import tpu_sc as plsc

_N = 10000
_D = 256
_H = 256
_C = 64
_E = 160000
_L = 2

_NC = 2
_NS = 16
_HH = _H // 2
_EPT = _E // _NS
_CH = 128
_EPP = 10240
_NCHUNK = _EPP // _CH
_NP = 10240
_RPT = _NP // _NS
_ZCH = 64
_NZ = _RPT // _ZCH
_IG = 16
_NG = _NCHUNK // _IG

_SRC_T = (0, 1, 1, 2)
_DST_T = (1, 0, 2, 1)

_BN = 1024
_NB = _NP // _BN


def _sc_segment_sums(h_stack, src_all, dst_all, zeros_hbm):
    mesh = plsc.VectorSubcoreMesh(core_axis_name="c", subcore_axis_name="s")
    NB = 8
    D = 6

    def body(h_ref, src_ref, dst_ref, z_ref, out_ref, *rest):
        src_v, dst_v = rest[0], rest[1]
        rows = rest[2:2 + NB]
        acc_sh = rest[2 + NB]
        gsem = rest[3 + NB:3 + 2 * NB]
        ssem = rest[3 + 2 * NB:3 + 3 * NB]
        c = lax.axis_index("c")
        s = lax.axis_index("s")
        row0 = s * _RPT

        for r in range(4):
            tab = h_ref.at[_SRC_T[r]]
            pltpu.sync_copy(z_ref, acc_sh.at[pl.ds(row0, _RPT)])
            pltpu.sync_copy(src_ref.at[r, s], src_v)
            pltpu.sync_copy(dst_ref.at[r, s], dst_v)
            plsc.subcore_barrier()

            def issue_g(ch, k):
                pltpu.async_copy(tab.at[c].at[src_v.at[ch]], rows[k],
                                 gsem[k])

            def wait_g(ch, k):
                pltpu.make_async_copy(tab.at[c].at[src_v.at[ch]], rows[k],
                                      gsem[k]).wait()

            def issue_s(ch, k):
                pltpu.async_copy(rows[k], acc_sh.at[dst_v.at[ch]], ssem[k],
                                 add=True)

            def drain_s(ch, k):
                pltpu.make_async_copy(rows[k], acc_sh.at[dst_v.at[ch]],
                                      ssem[k]).wait()

            for k in range(D):
                issue_g(k, k)
            for k in range(NB):
                kd = (k + D) % NB
                if k + D >= NB:
                    drain_s(k - D, kd)
                issue_g(k + D, kd)
                wait_g(k, k)
                issue_s(k, k)

            @pl.loop(1, _NCHUNK // NB - 1)
            def _octet(q):
                base = q * NB
                for k in range(NB):
                    ch = base + k
                    kd = (k + D) % NB
                    drain_s(ch - D, kd)
                    issue_g(ch + D, kd)
                    wait_g(ch, k)
                    issue_s(ch, k)

            base = _NCHUNK - NB
            for k in range(NB):
                ch = base + k
                if ch + D < _NCHUNK:
                    drain_s(ch - D, (ch + D) % NB)
                    issue_g(ch + D, (ch + D) % NB)
                wait_g(ch, k)
                issue_s(ch, k)
            for ch in range(_NCHUNK - D, _NCHUNK):
                drain_s(ch, ch % NB)
            for ch in range(_NCHUNK - NB, _NCHUNK - D):
                drain_s(ch, ch % NB)

            plsc.subcore_barrier()
            pltpu.sync_copy(acc_sh.at[pl.ds(row0, _RPT)],
                            out_ref.at[r, c, pl.ds(row0, _RPT)])

    scratch = [
        pltpu.VMEM((_NCHUNK, _CH), jnp.int32),
        pltpu.VMEM((_NCHUNK, _CH), jnp.int32),
    ]
    scratch += [pltpu.VMEM((_CH, _HH), jnp.bfloat16) for _ in range(NB)]
    scratch += [pltpu.VMEM_SHARED((_NP, _HH), jnp.bfloat16)]
    scratch += [pltpu.SemaphoreType.DMA for _ in range(2 * NB)]

    k = pl.kernel(
        body,
        out_type=jax.ShapeDtypeStruct((4, _NC, _NP, _HH), jnp.bfloat16),
        mesh=mesh,
        scratch_types=scratch,
        compiler_params=pltpu.CompilerParams(
            needs_layout_passes=False, use_tc_tiling_on_sc=False),
        name="sc_segsum",
    )
    return k(h_stack, src_all, dst_all, zeros_hbm)


def _sc_counts(dst_all):
    mesh = plsc.VectorSubcoreMesh(core_axis_name="c", subcore_axis_name="s")

    def body(dst_ref, cnt_ref, dst_v, hist_v, sem):
        c = lax.axis_index("c")
        s = lax.axis_index("s")
        zeros16 = jnp.zeros((16,), jnp.float32)
        ones16 = jnp.ones((16,), jnp.float32)
        for r in range(4):
            @pl.when(c == r // 2)
            def _rel():
                @pl.loop(0, _NP // 16)
                def _hz(i):
                    hist_v[pl.ds(i * 16, 16)] = zeros16

                for g in range(_NG):
                    pltpu.sync_copy(dst_ref.at[r, s, pl.ds(g * _IG, _IG)],
                                    dst_v)

                    @pl.loop(0, _IG)
                    def _chunk(j):
                        for gg in range(_CH // 16):
                            idx = dst_v[j, pl.ds(gg * 16, 16)]
                            plsc.addupdate_scatter(hist_v, [idx], ones16)

                pltpu.sync_copy(hist_v,
                                cnt_ref.at[pl.ds((s * 4 + r) * _NP, _NP)])

    return pl.kernel(
        body,
        out_type=jax.ShapeDtypeStruct((_NS * 4 * _NP,), jnp.float32),
        mesh=mesh,
        scratch_types=[
            pltpu.VMEM((_IG, _CH), jnp.int32),
            pltpu.VMEM((_NP,), jnp.float32),
            pltpu.SemaphoreType.DMA,
        ],
        compiler_params=pltpu.CompilerParams(
            needs_layout_passes=False, use_tc_tiling_on_sc=False),
        name="sc_counts",
    )(dst_all)


def _tc_project(xs, Wps, bps):
    def body(x_ref, w_ref, b_ref, o_ref):
        o_ref[0, 0] = (
            jnp.dot(x_ref[0], w_ref[0], preferred_element_type=jnp.float32)
            + b_ref[0, 0]
        ).astype(jnp.bfloat16)

    return pl.pallas_call(
        body,
        grid=(3, _NB, _NC),
        in_specs=[
            pl.BlockSpec((1, _BN, _D), lambda t, i, c: (t, i, 0)),
            pl.BlockSpec((1, _D, _HH), lambda t, i, c: (t, 0, c)),
            pl.BlockSpec((1, 1, 1, _HH), lambda t, i, c: (t, c, 0, 0)),
        ],
        out_specs=pl.BlockSpec((1, 1, _BN, _HH), lambda t, i, c: (t, c, i, 0)),
        out_shape=jax.ShapeDtypeStruct((3, _NC, _NP, _HH), jnp.bfloat16),
        name="tc_project",
    )(xs, Wps, bps)


def _tc_root(h_stack, Wr_l, bl_l):
    def body(h_ref, wr_ref, bl_ref, o_ref):
        for r in range(4):
            t = _DST_T[r]
            o_ref[r] = (
                jnp.dot(h_ref[t, 0].astype(jnp.float32), wr_ref[r, :_HH, :],
                        preferred_element_type=jnp.float32)
                + jnp.dot(h_ref[t, 1].astype(jnp.float32), wr_ref[r, _HH:, :],
                          preferred_element_type=jnp.float32)
                + bl_ref[r]
            )

    return pl.pallas_call(
        body,
        grid=(_NB,),
        in_specs=[
            pl.BlockSpec((3, _NC, _BN, _HH), lambda i: (0, 0, i, 0)),
            pl.BlockSpec((4, _H, _H), lambda i: (0, 0, 0)),
            pl.BlockSpec((4, _H), lambda i: (0, 0)),
        ],
        out_specs=pl.BlockSpec((4, _BN, _H), lambda i: (0, i, 0)),
        out_shape=jax.ShapeDtypeStruct((4, _NP, _H), jnp.float32),
        name="tc_root",
    )(h_stack, Wr_l, bl_l)


def _tc_layer_combine(sums, cnt, root, Wl_l):
    def body(s_ref, c_ref, rt_ref, wl_ref, o_ref):
        cnts = jnp.sum(c_ref[...], axis=0)
        o = []
        for r in range(4):
            acc = jnp.dot(s_ref[r, 0].astype(jnp.float32), wl_ref[r, :_HH, :],
                          preferred_element_type=jnp.float32)
            acc = acc + jnp.dot(s_ref[r, 1].astype(jnp.float32),
                                wl_ref[r, _HH:, :],
                                preferred_element_type=jnp.float32)
            inv = 1.0 / jnp.maximum(cnts[r][:, None], 1.0)
            o.append(acc * inv + rt_ref[r])
        new = (jnp.maximum(o[1], 0.0),
               jnp.maximum((o[0] + o[3]) * 0.5, 0.0),
               jnp.maximum(o[2], 0.0))
        for t in range(3):
            o_ref[t, 0] = new[t][:, :_HH].astype(jnp.bfloat16)
            o_ref[t, 1] = new[t][:, _HH:].astype(jnp.bfloat16)

    return pl.pallas_call(
        body,
        grid=(_NB,),
        in_specs=[
            pl.BlockSpec((4, _NC, _BN, _HH), lambda i: (0, 0, i, 0)),
            pl.BlockSpec((_NS, 4, _BN), lambda i: (0, 0, i)),
            pl.BlockSpec((4, _BN, _H), lambda i: (0, i, 0)),
            pl.BlockSpec((4, _H, _H), lambda i: (0, 0, 0)),
        ],
        out_specs=pl.BlockSpec((3, _NC, _BN, _HH), lambda i: (0, 0, i, 0)),
        out_shape=jax.ShapeDtypeStruct((3, _NC, _NP, _HH), jnp.bfloat16),
        name="tc_layer_combine",
    )(sums, cnt, root, Wl_l)


def _tc_classifier(h_stack, Wc, bc):
    def body(h_ref, w_ref, b_ref, o_ref):
        o_ref[...] = (
            jnp.dot(h_ref[0, 0].astype(jnp.float32), w_ref[:_HH, :],
                    preferred_element_type=jnp.float32)
            + jnp.dot(h_ref[0, 1].astype(jnp.float32), w_ref[_HH:, :],
                      preferred_element_type=jnp.float32)
            + b_ref[...]
        )

    return pl.pallas_call(
        body,
        grid=(_NB,),
        in_specs=[
            pl.BlockSpec((1, _NC, _BN, _HH), lambda i: (0, 0, i, 0)),
            pl.BlockSpec((_H, _C), lambda i: (0, 0)),
            pl.BlockSpec((1, _C), lambda i: (0, 0)),
        ],
        out_specs=pl.BlockSpec((_BN, _C), lambda i: (i, 0)),
        out_shape=jax.ShapeDtypeStruct((_NP, _C), jnp.float32),
        name="tc_classifier",
    )(h_stack, Wc, bc.reshape(1, _C))


def kernel(x_lncRNA, x_miRNA, x_mRNA, edge_index_interacts,
           edge_index_rev_interacts, edge_index_regulates,
           edge_index_rev_regulates, Wp_lncRNA, bp_lncRNA, Wp_miRNA, bp_miRNA,
           Wp_mRNA, bp_mRNA, Wl, bl, Wr, Wc, bc):
    xs = jnp.stack([x_lncRNA, x_miRNA, x_mRNA])
    xs = jnp.pad(xs, ((0, 0), (0, _NP - _N), (0, 0)))
    Wps = jnp.stack([Wp_lncRNA, Wp_miRNA, Wp_mRNA])
    bps = jnp.stack([bp_lncRNA, bp_miRNA, bp_mRNA]).reshape(3, _NC, 1, _HH)

    eis = (edge_index_interacts, edge_index_rev_interacts,
           edge_index_regulates, edge_index_rev_regulates)
    pad_row = jnp.int32(_NP - 1)

    def _prep(row):
        row = row.reshape(_NS, _EPT)
        row = jnp.pad(row, ((0, 0), (0, _EPP - _EPT)), constant_values=pad_row)
        return row.reshape(_NS, _NCHUNK, _CH)

    src_all = jnp.stack([_prep(e[0]) for e in eis])
    dst_all = jnp.stack([_prep(e[1]) for e in eis])

    h = _tc_project(xs, Wps, bps)

    cnt = _sc_counts(dst_all).reshape(_NS, 4, _NP)
    zeros_hbm = jnp.zeros((_RPT, _HH), jnp.bfloat16)
    for l in range(_L):
        sums = _sc_segment_sums(h, src_all, dst_all, zeros_hbm)
        root = _tc_root(h, Wr[l], bl[l])
        h = _tc_layer_combine(sums, cnt, root, Wl[l])

    return _tc_classifier(h, Wc, bc)[:_N]
